# Initial kernel scaffold; baseline (speedup 1.0000x reference)
import jax
import jax.numpy as jnp
from jax.experimental import pallas as pl


def kernel(x, edge_index, edge_attr, W1, b1, W2, b2, W3, b3):
    raise NotImplementedError("write your pallas kernel here")


# speedup vs baseline: 5.4349x; 5.4349x over previous
import functools

import jax
import jax.numpy as jnp
from jax import lax
from jax.experimental import pallas as pl
from jax.experimental.pallas import tpu as pltpu
from jax.experimental.pallas import tpu_sc as plsc

N = 10000
F = 128
E = 320000

NC = 2
NS = 16
NW = NC * NS
EPT = E // NW
CH = 80
CPT = EPT // CH
FH = F // 2
EPS = E // NS
CPS = EPS // CH
RPT_OFF = 624
RPT_LEN = 640

_sc_mesh = plsc.VectorSubcoreMesh(core_axis_name="c", subcore_axis_name="s")


@functools.partial(
    pl.kernel,
    out_type=jax.ShapeDtypeStruct((NC * N,), jnp.float32),
    mesh=_sc_mesh,
    scratch_types=[
        pltpu.VMEM((CH,), jnp.int32),
        pltpu.VMEM((CH,), jnp.float32),
        pltpu.VMEM((RPT_LEN,), jnp.float32),
        pltpu.VMEM_SHARED((N,), jnp.float32),
    ],
)
def _sc_deg(dst_hbm, w_hbm, zflat_hbm, out_hbm, idxv, wv, zv, acc):
    c = lax.axis_index("c")
    s = lax.axis_index("s")
    wid = s * NC + c
    pltpu.sync_copy(zflat_hbm, zv)
    pltpu.sync_copy(zv, acc.at[pl.ds(s * RPT_OFF, RPT_LEN)])
    plsc.subcore_barrier()

    def body(i, carry):
        base = wid * EPT + i * CH
        pltpu.sync_copy(dst_hbm.at[pl.ds(base, CH)], idxv)
        pltpu.sync_copy(w_hbm.at[pl.ds(base, CH)], wv)
        pltpu.sync_copy(wv, acc.at[idxv], add=True)
        return carry

    lax.fori_loop(0, CPT, body, 0)
    plsc.subcore_barrier()
    pltpu.sync_copy(acc.at[pl.ds(s * RPT_OFF, RPT_LEN)], zv)
    pltpu.sync_copy(zv, out_hbm.at[pl.ds(c * N + s * RPT_OFF, RPT_LEN)])


@functools.partial(
    pl.kernel,
    out_type=[
        jax.ShapeDtypeStruct((N, FH), jnp.float32),
        jax.ShapeDtypeStruct((N, FH), jnp.float32),
    ],
    mesh=_sc_mesh,
    scratch_types=[
        pltpu.VMEM((CH,), jnp.int32),
        pltpu.VMEM((CH,), jnp.int32),
        pltpu.VMEM((CH,), jnp.float32),
        pltpu.VMEM((CH, FH), jnp.float32),
        pltpu.VMEM((RPT_LEN, FH), jnp.float32),
        pltpu.VMEM_SHARED((N, FH), jnp.float32),
        pltpu.SemaphoreType.DMA,
    ],
    compiler_params=pltpu.CompilerParams(use_tc_tiling_on_sc=False),
)
def _sc_scat(ga_hbm, gb_hbm, src_hbm, dst_hbm, w_hbm, zrow_hbm,
             outa_hbm, outb_hbm, srcv, dstv, wv, rows, vbuf, acc, sem):
    c = lax.axis_index("c")
    s = lax.axis_index("s")
    pltpu.sync_copy(zrow_hbm, vbuf)
    pltpu.sync_copy(vbuf, acc.at[pl.ds(s * RPT_OFF, RPT_LEN)])
    plsc.subcore_barrier()

    def body(i, carry):
        base = s * EPS + i * CH
        pltpu.sync_copy(src_hbm.at[pl.ds(base, CH)], srcv)
        pltpu.sync_copy(dst_hbm.at[pl.ds(base, CH)], dstv)
        pltpu.sync_copy(w_hbm.at[pl.ds(base, CH)], wv)

        @pl.when(c == 0)
        def _():
            pltpu.async_copy(ga_hbm.at[srcv], rows, sem).wait()

        @pl.when(c == 1)
        def _():
            pltpu.async_copy(gb_hbm.at[srcv], rows, sem).wait()

        for k in range(CH // 16):
            nv = wv[pl.ds(k * 16, 16)]
            for t in range(16):
                e = k * 16 + t
                sc = nv[t]
                for j in range(FH // 16):
                    rows[e, pl.ds(j * 16, 16)] = rows[e, pl.ds(j * 16, 16)] * sc
        pltpu.sync_copy(rows, acc.at[dstv], add=True)
        return carry

    lax.fori_loop(0, CPS, body, 0)
    plsc.subcore_barrier()
    pltpu.sync_copy(acc.at[pl.ds(s * RPT_OFF, RPT_LEN)], vbuf)

    @pl.when(c == 0)
    def _():
        pltpu.sync_copy(vbuf, outa_hbm.at[pl.ds(s * RPT_OFF, RPT_LEN)])

    @pl.when(c == 1)
    def _():
        pltpu.sync_copy(vbuf, outb_hbm.at[pl.ds(s * RPT_OFF, RPT_LEN)])


_R = 1000


def _tc1_body(degT_ref, x_ref, w_ref, dinv_ref, ga_ref, gb_ref):
    p = degT_ref[...]
    deg = jnp.sum(p, axis=1, keepdims=True) + 1.0
    dinv = lax.rsqrt(deg)
    dinv_ref[...] = dinv
    h = jnp.dot(x_ref[...], w_ref[...], preferred_element_type=jnp.float32)
    g = dinv * h
    ga_ref[...] = g[:, :FH]
    gb_ref[...] = g[:, FH:]


_tc1 = pl.pallas_call(
    _tc1_body,
    grid=(N // _R,),
    in_specs=[
        pl.BlockSpec((_R, NC), lambda i: (i, 0)),
        pl.BlockSpec((_R, F), lambda i: (i, 0)),
        pl.BlockSpec((F, F), lambda i: (0, 0)),
    ],
    out_specs=[
        pl.BlockSpec((_R, 1), lambda i: (i, 0)),
        pl.BlockSpec((_R, FH), lambda i: (i, 0)),
        pl.BlockSpec((_R, FH), lambda i: (i, 0)),
    ],
    out_shape=[
        jax.ShapeDtypeStruct((N, 1), jnp.float32),
        jax.ShapeDtypeStruct((N, FH), jnp.float32),
        jax.ShapeDtypeStruct((N, FH), jnp.float32),
    ],
)


def _tc_mid_body(sa_ref, sb_ref, gpa_ref, gpb_ref, dinv_ref, b_ref, w_ref,
                 z_ref, ga_ref, gb_ref):
    d = dinv_ref[...]
    spg = jnp.concatenate(
        [sa_ref[...] + gpa_ref[...], sb_ref[...] + gpb_ref[...]], axis=1)
    z = d * spg + b_ref[...]
    z_ref[...] = z
    a = jnp.maximum(z, 0.0)
    g = d * jnp.dot(a, w_ref[...], preferred_element_type=jnp.float32)
    ga_ref[...] = g[:, :FH]
    gb_ref[...] = g[:, FH:]


_tc_mid = pl.pallas_call(
    _tc_mid_body,
    grid=(N // _R,),
    in_specs=[
        pl.BlockSpec((_R, FH), lambda i: (i, 0)),
        pl.BlockSpec((_R, FH), lambda i: (i, 0)),
        pl.BlockSpec((_R, FH), lambda i: (i, 0)),
        pl.BlockSpec((_R, FH), lambda i: (i, 0)),
        pl.BlockSpec((_R, 1), lambda i: (i, 0)),
        pl.BlockSpec((1, F), lambda i: (0, 0)),
        pl.BlockSpec((F, F), lambda i: (0, 0)),
    ],
    out_specs=[
        pl.BlockSpec((_R, F), lambda i: (i, 0)),
        pl.BlockSpec((_R, FH), lambda i: (i, 0)),
        pl.BlockSpec((_R, FH), lambda i: (i, 0)),
    ],
    out_shape=[
        jax.ShapeDtypeStruct((N, F), jnp.float32),
        jax.ShapeDtypeStruct((N, FH), jnp.float32),
        jax.ShapeDtypeStruct((N, FH), jnp.float32),
    ],
)


def _tc_out_body(z_ref, out_ref):
    z = z_ref[...]
    m = jnp.max(z, axis=1, keepdims=True)
    zc = z - m
    lse = jnp.log(jnp.sum(jnp.exp(zc), axis=1, keepdims=True))
    out_ref[...] = zc - lse


_tc_out = pl.pallas_call(
    _tc_out_body,
    grid=(N // _R,),
    in_specs=[pl.BlockSpec((_R, F), lambda i: (i, 0))],
    out_specs=pl.BlockSpec((_R, F), lambda i: (i, 0)),
    out_shape=jax.ShapeDtypeStruct((N, F), jnp.float32),
)


def kernel(x, edge_index, edge_attr, W1, b1, W2, b2, W3, b3):
    src = edge_index[0].astype(jnp.int32)
    dst = edge_index[1].astype(jnp.int32)
    w = edge_attr.astype(jnp.float32)
    zflat = jnp.zeros((RPT_LEN,), jnp.float32)
    zrow = jnp.zeros((RPT_LEN, FH), jnp.float32)

    degp = _sc_deg(dst, w, zflat)
    degT = degp.reshape(NC, N).T
    dinv, g1a, g1b = _tc1(degT, x, W1)

    bs = jnp.stack([b1, b2, b3]).reshape(3, 1, F)
    Ws = jnp.stack([W2, W3, W2])

    def step(carry, xs):
        ga, gb, _ = carry
        b, W = xs
        sa, sb = _sc_scat(ga, gb, src, dst, w, zrow)
        z, gna, gnb = _tc_mid(sa, sb, ga, gb, dinv, b, W)
        return (gna, gnb, z), None

    z0 = jnp.zeros((N, F), jnp.float32)
    (_, _, z3), _ = lax.scan(step, (g1a, g1b, z0), (bs, Ws))
    return _tc_out(z3)
